# baseline (speedup 1.0000x reference)
import functools

import jax
import jax.numpy as jnp
from jax.experimental import pallas as pl
from jax.experimental.pallas import tpu as pltpu

N = 16384
D = 64
K = 1024
BLK = 512
GRID = N // BLK


def _vq_body(x_ref, w_ref, q_ref, idx_ref, loss_ref, perp_ref,
             cnt_ref, acc_ref):
    i = pl.program_id(0)

    @pl.when(i == 0)
    def _init():
        cnt_ref[...] = jnp.zeros_like(cnt_ref)
        acc_ref[0, 0] = jnp.float32(0.0)

    x = x_ref[...]
    w = w_ref[...]

    mm = jax.lax.dot_general(
        x, w, (((1,), (1,)), ((), ())),
        preferred_element_type=jnp.float32)
    xsq = jnp.sum(x * x, axis=1, keepdims=True)
    wsq = jnp.sum(w * w, axis=1)
    dist = (xsq + wsq[None, :]) - 2.0 * mm

    m = jnp.min(dist, axis=1, keepdims=True)
    iota = jax.lax.broadcasted_iota(jnp.int32, (BLK, K), 1)
    cand = jnp.where(dist == m, iota, K)
    idx2 = jnp.min(cand, axis=1, keepdims=True)
    idx_ref[...] = idx2.reshape(BLK)

    enc = (iota == idx2).astype(jnp.float32)
    q = jax.lax.dot_general(
        enc, w, (((1,), (0,)), ((), ())),
        preferred_element_type=jnp.float32,
        precision=jax.lax.Precision.HIGHEST)
    q_ref[...] = x + (q - x)

    diff = q - x
    acc_ref[0, 0] += jnp.sum(diff * diff)
    cnt_ref[...] += jnp.sum(enc, axis=0, keepdims=True)

    @pl.when(i == GRID - 1)
    def _fin():
        loss_ref[0, 0] = acc_ref[0, 0] * (0.25 / (N * D))
        p = cnt_ref[...] * (1.0 / N)
        perp_ref[0, 0] = jnp.exp(-jnp.sum(p * jnp.log(p + 1e-10)))


@functools.partial(jax.jit, static_argnames=("interpret",))
def _vq_call(flat, W, interpret=False):
    q, idx, loss, perp = pl.pallas_call(
        _vq_body,
        grid=(GRID,),
        in_specs=[
            pl.BlockSpec((BLK, D), lambda i: (i, 0)),
            pl.BlockSpec((K, D), lambda i: (0, 0)),
        ],
        out_specs=[
            pl.BlockSpec((BLK, D), lambda i: (i, 0)),
            pl.BlockSpec((BLK,), lambda i: (i,)),
            pl.BlockSpec(memory_space=pltpu.SMEM),
            pl.BlockSpec(memory_space=pltpu.SMEM),
        ],
        out_shape=[
            jax.ShapeDtypeStruct((N, D), jnp.float32),
            jax.ShapeDtypeStruct((N,), jnp.int32),
            jax.ShapeDtypeStruct((1, 1), jnp.float32),
            jax.ShapeDtypeStruct((1, 1), jnp.float32),
        ],
        scratch_shapes=[
            pltpu.VMEM((1, K), jnp.float32),
            pltpu.SMEM((1, 1), jnp.float32),
        ],
        interpret=interpret,
    )(flat, W)
    return q, idx, loss, perp


def kernel(inputs, W):
    input_shape = inputs.shape
    flat = inputs.reshape(-1, D)
    q, idx, loss, perp = _vq_call(flat, W)
    return (q.reshape(input_shape), loss[0, 0], idx, perp[0, 0])


# speedup vs baseline: 1.2187x; 1.2187x over previous
import functools

import jax
import jax.numpy as jnp
from jax import lax
from jax.experimental import pallas as pl
from jax.experimental.pallas import tpu as pltpu
from jax.experimental.pallas import tpu_sc as plsc

N = 16384
D = 64
K = 1024
BLK = 512
GRID = N // BLK


def _vq_body(x_ref, w_ref, idx_ref, loss_ref, perp_ref, cnt_ref, acc_ref):
    i = pl.program_id(0)

    @pl.when(i == 0)
    def _init():
        cnt_ref[...] = jnp.zeros_like(cnt_ref)
        acc_ref[0, 0] = jnp.float32(0.0)

    x = x_ref[...]
    w = w_ref[...]

    mm = jax.lax.dot_general(
        x, w, (((1,), (1,)), ((), ())),
        preferred_element_type=jnp.float32)
    xsq = jnp.sum(x * x, axis=1, keepdims=True)
    wsq = jnp.sum(w * w, axis=1)
    dist = (xsq + wsq[None, :]) - 2.0 * mm

    m = jnp.min(dist, axis=1, keepdims=True)
    iota = jax.lax.broadcasted_iota(jnp.int32, (BLK, K), 1)
    cand = jnp.where(dist == m, iota, K)
    idx2 = jnp.min(cand, axis=1, keepdims=True)
    idx_ref[...] = idx2.reshape(BLK)

    acc_ref[0, 0] += jnp.sum(m)
    enc = (iota == idx2).astype(jnp.float32)
    cnt_ref[...] += jnp.sum(enc, axis=0, keepdims=True)

    @pl.when(i == GRID - 1)
    def _fin():
        loss_ref[0, 0] = acc_ref[0, 0] * (0.25 / (N * D))
        p = cnt_ref[...] * (1.0 / N)
        perp_ref[0, 0] = jnp.exp(-jnp.sum(p * jnp.log(p + 1e-10)))


def _vq_tc(flat, W):
    return pl.pallas_call(
        _vq_body,
        grid=(GRID,),
        in_specs=[
            pl.BlockSpec((BLK, D), lambda i: (i, 0)),
            pl.BlockSpec((K, D), lambda i: (0, 0)),
        ],
        out_specs=[
            pl.BlockSpec((BLK,), lambda i: (i,)),
            pl.BlockSpec(memory_space=pltpu.SMEM),
            pl.BlockSpec(memory_space=pltpu.SMEM),
        ],
        out_shape=[
            jax.ShapeDtypeStruct((N,), jnp.int32),
            jax.ShapeDtypeStruct((1, 1), jnp.float32),
            jax.ShapeDtypeStruct((1, 1), jnp.float32),
        ],
        scratch_shapes=[
            pltpu.VMEM((1, K), jnp.float32),
            pltpu.SMEM((1, 1), jnp.float32),
        ],
    )(flat, W)


_info = plsc.get_sparse_core_info()
_NC, _NS = _info.num_cores, _info.num_subcores
_NW = _NC * _NS
_BPW = N // _NW
_CHUNK = 128
_NCHUNK = _BPW // _CHUNK
_DP = 128


def _make_sc_gather():
    mesh = plsc.VectorSubcoreMesh(core_axis_name="c", subcore_axis_name="s")

    @functools.partial(
        pl.kernel, mesh=mesh,
        out_type=jax.ShapeDtypeStruct((N, _DP), jnp.float32),
        scratch_types=[
            pltpu.VMEM((_NCHUNK, _CHUNK), jnp.int32),
            pltpu.VMEM((_BPW, _DP), jnp.float32),
            pltpu.SemaphoreType.DMA,
        ],
    )
    def gather_k(w_hbm, idx_hbm, out_hbm, idx_v, rows_v, sem):
        wid = lax.axis_index("s") * _NC + lax.axis_index("c")
        pltpu.sync_copy(idx_hbm.at[pl.ds(wid * _NCHUNK, _NCHUNK)], idx_v)
        handles = [
            pltpu.async_copy(w_hbm.at[idx_v.at[j]],
                             rows_v.at[pl.ds(j * _CHUNK, _CHUNK)], sem)
            for j in range(_NCHUNK)
        ]
        for h in handles:
            h.wait()
        pltpu.sync_copy(rows_v, out_hbm.at[pl.ds(wid * _BPW, _BPW)])

    return gather_k


_sc_gather = _make_sc_gather()


def kernel(inputs, W):
    input_shape = inputs.shape
    flat = inputs.reshape(-1, D)
    idx, loss, perp = _vq_tc(flat, W)
    w_pad = jnp.concatenate([W, jnp.zeros((K, _DP - D), jnp.float32)], axis=1)
    q = _sc_gather(w_pad, idx.reshape(N // _CHUNK, _CHUNK))[:, :D]
    return (q.reshape(input_shape), loss[0, 0], idx, perp[0, 0])


# speedup vs baseline: 1.4056x; 1.1534x over previous
import functools

import jax
import jax.numpy as jnp
from jax import lax
from jax.experimental import pallas as pl
from jax.experimental.pallas import tpu as pltpu
from jax.experimental.pallas import tpu_sc as plsc

N = 16384
D = 64
K = 1024
BLK = 512
GRID = N // BLK


def _vq_body(x_ref, wt_ref, idx_ref, loss_ref, perp_ref,
             cnt_ref, acc_ref, wsq_ref, iotaf_ref):
    i = pl.program_id(0)

    @pl.when(i == 0)
    def _init():
        cnt_ref[...] = jnp.zeros_like(cnt_ref)
        acc_ref[0, 0] = jnp.float32(0.0)
        wt0 = wt_ref[...]
        wsq_ref[...] = jnp.sum(wt0 * wt0, axis=0, keepdims=True)
        iotaf_ref[...] = jax.lax.broadcasted_iota(
            jnp.int32, (1, K), 1).astype(jnp.float32)

    x = x_ref[...]
    wt = wt_ref[...]

    mm = jax.lax.dot_general(
        x, wt, (((1,), (0,)), ((), ())),
        preferred_element_type=jnp.float32)
    xsq = jnp.sum(x * x, axis=1, keepdims=True)
    dist = (xsq + wsq_ref[...]) - 2.0 * mm

    m = jnp.min(dist, axis=1, keepdims=True)
    eq = dist == m
    cand = jnp.where(eq, iotaf_ref[...], jnp.float32(K))
    idxf = jnp.min(cand, axis=1, keepdims=True)
    idx_ref[...] = idxf.astype(jnp.int32)

    acc_ref[0, 0] += jnp.sum(m)
    cnt_ref[...] += jnp.sum(jnp.where(eq, 1.0, 0.0), axis=0, keepdims=True)

    @pl.when(i == GRID - 1)
    def _fin():
        loss_ref[0, 0] = acc_ref[0, 0] * (0.25 / (N * D))
        p = cnt_ref[...] * (1.0 / N)
        perp_ref[0, 0] = jnp.exp(-jnp.sum(p * jnp.log(p + 1e-10)))


def _vq_tc(flat, WT):
    return pl.pallas_call(
        _vq_body,
        grid=(GRID,),
        in_specs=[
            pl.BlockSpec((BLK, D), lambda i: (i, 0)),
            pl.BlockSpec((D, K), lambda i: (0, 0)),
        ],
        out_specs=[
            pl.BlockSpec((BLK, 1), lambda i: (i, 0)),
            pl.BlockSpec(memory_space=pltpu.SMEM),
            pl.BlockSpec(memory_space=pltpu.SMEM),
        ],
        out_shape=[
            jax.ShapeDtypeStruct((N, 1), jnp.int32),
            jax.ShapeDtypeStruct((1, 1), jnp.float32),
            jax.ShapeDtypeStruct((1, 1), jnp.float32),
        ],
        scratch_shapes=[
            pltpu.VMEM((1, K), jnp.float32),
            pltpu.SMEM((1, 1), jnp.float32),
            pltpu.VMEM((1, K), jnp.float32),
            pltpu.VMEM((1, K), jnp.float32),
        ],
    )(flat, WT)


_info = plsc.get_sparse_core_info()
_NC, _NS = _info.num_cores, _info.num_subcores
_NW = _NC * _NS
_BPW = N // _NW
_CHUNK = 128
_NCHUNK = _BPW // _CHUNK
_DP = 128


def _make_sc_gather():
    mesh = plsc.VectorSubcoreMesh(core_axis_name="c", subcore_axis_name="s")

    @functools.partial(
        pl.kernel, mesh=mesh,
        out_type=jax.ShapeDtypeStruct((N, _DP), jnp.float32),
        scratch_types=[
            pltpu.VMEM((_NCHUNK, _CHUNK), jnp.int32),
            pltpu.VMEM((_BPW, _DP), jnp.float32),
            pltpu.SemaphoreType.DMA,
        ],
    )
    def gather_k(w_hbm, idx_hbm, out_hbm, idx_v, rows_v, sem):
        wid = lax.axis_index("s") * _NC + lax.axis_index("c")
        pltpu.sync_copy(idx_hbm.at[pl.ds(wid * _NCHUNK, _NCHUNK)], idx_v)
        handles = [
            pltpu.async_copy(w_hbm.at[idx_v.at[j]],
                             rows_v.at[pl.ds(j * _CHUNK, _CHUNK)], sem)
            for j in range(_NCHUNK)
        ]
        for h in handles:
            h.wait()
        pltpu.sync_copy(rows_v, out_hbm.at[pl.ds(wid * _BPW, _BPW)])

    return gather_k


_sc_gather = _make_sc_gather()


def kernel(inputs, W):
    input_shape = inputs.shape
    flat = inputs.reshape(-1, D)
    idx2d, loss, perp = _vq_tc(flat, W.T)
    idx = idx2d.reshape(N)
    w_pad = jnp.concatenate([W, jnp.zeros((K, _DP - D), jnp.float32)], axis=1)
    q = _sc_gather(w_pad, idx.reshape(N // _CHUNK, _CHUNK))[:, :D]
    return (q.reshape(input_shape), loss[0, 0], idx, perp[0, 0])


# speedup vs baseline: 1.8986x; 1.3507x over previous
import functools

import jax
import jax.numpy as jnp
from jax import lax
from jax.experimental import pallas as pl
from jax.experimental.pallas import tpu as pltpu
from jax.experimental.pallas import tpu_sc as plsc

N = 16384
D = 64
K = 1024
BLK = 512
GRID = N // BLK


def _vq_body(x_ref, wt_ref, w_ref, q_ref, idx_ref, loss_ref, perp_ref,
             cnt_ref, acc_ref, wsq_ref, iotaf_ref):
    i = pl.program_id(0)

    @pl.when(i == 0)
    def _init():
        cnt_ref[...] = jnp.zeros_like(cnt_ref)
        acc_ref[0, 0] = jnp.float32(0.0)
        wt0 = wt_ref[...]
        wsq_ref[...] = jnp.sum(wt0 * wt0, axis=0, keepdims=True)
        iotaf_ref[...] = jax.lax.broadcasted_iota(
            jnp.int32, (1, K), 1).astype(jnp.float32)

    x = x_ref[...]
    wt = wt_ref[...]

    mm = jax.lax.dot_general(
        x, wt, (((1,), (0,)), ((), ())),
        preferred_element_type=jnp.float32)
    xsq = jnp.sum(x * x, axis=1, keepdims=True)
    dist = (xsq + wsq_ref[...]) - 2.0 * mm

    m = jnp.min(dist, axis=1, keepdims=True)
    eq = dist == m
    cand = jnp.where(eq, iotaf_ref[...], jnp.float32(K))
    idxf = jnp.min(cand, axis=1, keepdims=True)
    idx_ref[...] = idxf.astype(jnp.int32)

    enc = jnp.where(iotaf_ref[...] == idxf, 1.0, 0.0)
    q = jax.lax.dot_general(
        enc, w_ref[...], (((1,), (0,)), ((), ())),
        preferred_element_type=jnp.float32)
    q_ref[...] = x + (q - x)

    acc_ref[0, 0] += jnp.sum(m)
    cnt_ref[...] += jnp.sum(enc, axis=0, keepdims=True)
    del eq

    @pl.when(i == GRID - 1)
    def _fin():
        loss_ref[0, 0] = acc_ref[0, 0] * (0.25 / (N * D))
        p = cnt_ref[...] * (1.0 / N)
        perp_ref[0, 0] = jnp.exp(-jnp.sum(p * jnp.log(p + 1e-10)))


def _vq_tc(flat, WT, W):
    return pl.pallas_call(
        _vq_body,
        grid=(GRID,),
        in_specs=[
            pl.BlockSpec((BLK, D), lambda i: (i, 0)),
            pl.BlockSpec((D, K), lambda i: (0, 0)),
            pl.BlockSpec((K, D), lambda i: (0, 0)),
        ],
        out_specs=[
            pl.BlockSpec((BLK, D), lambda i: (i, 0)),
            pl.BlockSpec((BLK, 1), lambda i: (i, 0)),
            pl.BlockSpec(memory_space=pltpu.SMEM),
            pl.BlockSpec(memory_space=pltpu.SMEM),
        ],
        out_shape=[
            jax.ShapeDtypeStruct((N, D), jnp.float32),
            jax.ShapeDtypeStruct((N, 1), jnp.int32),
            jax.ShapeDtypeStruct((1, 1), jnp.float32),
            jax.ShapeDtypeStruct((1, 1), jnp.float32),
        ],
        scratch_shapes=[
            pltpu.VMEM((1, K), jnp.float32),
            pltpu.SMEM((1, 1), jnp.float32),
            pltpu.VMEM((1, K), jnp.float32),
            pltpu.VMEM((1, K), jnp.float32),
        ],
    )(flat, WT, W)


_info = plsc.get_sparse_core_info()
_NC, _NS = _info.num_cores, _info.num_subcores
_NW = _NC * _NS
_BPW = N // _NW
_CHUNK = 128
_NCHUNK = _BPW // _CHUNK
_DP = 128


def _make_sc_gather():
    mesh = plsc.VectorSubcoreMesh(core_axis_name="c", subcore_axis_name="s")

    @functools.partial(
        pl.kernel, mesh=mesh,
        out_type=jax.ShapeDtypeStruct((N, _DP), jnp.float32),
        scratch_types=[
            pltpu.VMEM((_NCHUNK, _CHUNK), jnp.int32),
            pltpu.VMEM((_BPW, _DP), jnp.float32),
            pltpu.SemaphoreType.DMA,
        ],
    )
    def gather_k(w_hbm, idx_hbm, out_hbm, idx_v, rows_v, sem):
        wid = lax.axis_index("s") * _NC + lax.axis_index("c")
        pltpu.sync_copy(idx_hbm.at[pl.ds(wid * _NCHUNK, _NCHUNK)], idx_v)
        handles = [
            pltpu.async_copy(w_hbm.at[idx_v.at[j]],
                             rows_v.at[pl.ds(j * _CHUNK, _CHUNK)], sem)
            for j in range(_NCHUNK)
        ]
        for h in handles:
            h.wait()
        pltpu.sync_copy(rows_v, out_hbm.at[pl.ds(wid * _BPW, _BPW)])

    return gather_k


_sc_gather = _make_sc_gather()


def kernel(inputs, W):
    input_shape = inputs.shape
    flat = inputs.reshape(-1, D)
    q, idx2d, loss, perp = _vq_tc(flat, W.T, W)
    idx = idx2d.reshape(N)
    return (q.reshape(input_shape), loss[0, 0], idx, perp[0, 0])
